# baseline (speedup 1.0000x reference)
import functools

import jax
import jax.numpy as jnp
from jax import lax
from jax.experimental import pallas as pl
from jax.experimental.pallas import tpu as pltpu
from jax.experimental.pallas import tpu_sc as plsc

NUM_CORES = 2
NUM_SUBCORES = 16
NUM_WORKERS = NUM_CORES * NUM_SUBCORES
LANES = 16
DIM = 64
SCALE = 8.0

VOCAB = 1000000
PAIR_ROWS = VOCAB // 2
TCOLS = 256
TROWS = TCOLS // 2
NBLK = VOCAB // TCOLS
FULL_COLS = NBLK * TCOLS
TAIL_ROWS = (VOCAB - FULL_COLS) // 2
NBUF1 = 3
STEPS1 = 126

CHUNK = 128
NBUF2 = 4


def _iota16():
    return lax.iota(jnp.int32, LANES)


@functools.lru_cache(maxsize=None)
def _make_retile():
    mesh = plsc.VectorSubcoreMesh(core_axis_name="c", subcore_axis_name="s")

    @functools.partial(
        pl.kernel,
        mesh=mesh,
        out_type=jax.ShapeDtypeStruct((PAIR_ROWS, 2 * DIM), jnp.float32),
        scratch_types=(
            [pltpu.VMEM((DIM, TCOLS), jnp.float32)] * NBUF1
            + [pltpu.VMEM((TROWS, 2 * DIM), jnp.float32)] * NBUF1
            + [pltpu.SemaphoreType.DMA] * (2 * NBUF1)
        ),
        compiler_params=pltpu.CompilerParams(needs_layout_passes=False,
                                             disable_bounds_checks=True),
    )
    def retile(tt_hbm, tail_hbm, s_hbm, in0, in1, in2, tp0, tp1, tp2,
               is0, is1, is2, os0, os1, os2):
        wid = lax.axis_index("s") * NUM_CORES + lax.axis_index("c")
        inbufs = (in0, in1, in2)
        tpbufs = (tp0, tp1, tp2)
        isems = (is0, is1, is2)
        osems = (os0, os1, os2)
        dvecs = [_iota16() + d0 for d0 in range(0, DIM, LANES)]

        def block_id(k):
            b = wid + NUM_WORKERS * k
            return jnp.where(b < NBLK, b, b - NBLK)

        def in_slice(k):
            c0 = pl.multiple_of(block_id(k) * TCOLS, TCOLS)
            return tt_hbm.at[:, pl.ds(c0, TCOLS)]

        def out_slice(k):
            r0 = pl.multiple_of(block_id(k) * TROWS, TROWS)
            return s_hbm.at[pl.ds(r0, TROWS)]

        def issue_in(k, par):
            pltpu.async_copy(in_slice(k), inbufs[par], isems[par])

        def wait_in(k, par):
            pltpu.make_async_copy(in_slice(k), inbufs[par], isems[par]).wait()

        def issue_out(k, par):
            pltpu.async_copy(tpbufs[par], out_slice(k), osems[par])

        def wait_out(k, par):
            pltpu.make_async_copy(tpbufs[par], out_slice(k), osems[par]).wait()

        def transpose_block(par):
            src = inbufs[par]
            dst = tpbufs[par]

            @plsc.parallel_loop(0, TROWS, step=1, unroll=2)
            def _row(r):
                cv0 = jnp.broadcast_to(2 * r, (LANES,)).astype(jnp.int32)
                cv1 = cv0 + 1
                for h, cv in ((0, cv0), (1, cv1)):
                    for q in range(DIM // LANES):
                        v = plsc.load_gather(src, [dvecs[q], cv])
                        dst[r, pl.ds(h * DIM + q * LANES, LANES)] = v

        def step(k, par, first):
            issue_in(k + 2, (par + 2) % NBUF1)
            wait_in(k, par)
            if not first:
                wait_out(k - NBUF1, par)
            transpose_block(par)
            issue_out(k, par)

        issue_in(0, 0)
        issue_in(1, 1)
        for k in range(NBUF1):
            step(k, k, True)

        def fori_body(i, carry):
            k0 = NBUF1 * i
            for j in range(NBUF1):
                step(k0 + j, j, False)
            return carry

        lax.fori_loop(1, STEPS1 // NBUF1, fori_body, 0)

        wait_in(STEPS1, STEPS1 % NBUF1)
        wait_in(STEPS1 + 1, (STEPS1 + 1) % NBUF1)
        for j in range(NBUF1):
            k = STEPS1 - NBUF1 + j
            wait_out(k, k % NBUF1)

        @pl.when(wid == NUM_WORKERS - 1)
        def _tail():
            pltpu.sync_copy(tail_hbm, tp0.at[pl.ds(0, TAIL_ROWS)])
            pltpu.sync_copy(tp0.at[pl.ds(0, TAIL_ROWS)],
                            s_hbm.at[pl.ds(PAIR_ROWS - TAIL_ROWS, TAIL_ROWS)])

    return retile


@functools.lru_cache(maxsize=None)
def _make_lookup(n_chunks: int, n_sent: int, sent_len: int):
    mesh = plsc.VectorSubcoreMesh(core_axis_name="c", subcore_axis_name="s")
    blocks_per_sent = sent_len // CHUNK

    @functools.partial(
        pl.kernel,
        mesh=mesh,
        out_type=jax.ShapeDtypeStruct(
            (n_sent, DIM // 8, sent_len // CHUNK, 8, CHUNK), jnp.float32),
        scratch_types=(
            [pltpu.VMEM((n_chunks, CHUNK), jnp.int32)]
            + [pltpu.VMEM((CHUNK, DIM), jnp.float32)] * NBUF2
            + [pltpu.VMEM((DIM // 8, 8, CHUNK), jnp.float32)] * 2
            + [pltpu.SemaphoreType.DMA] * (NBUF2 + 2)
        ),
        compiler_params=pltpu.CompilerParams(use_tc_tiling_on_sc=False,
                                             needs_layout_passes=False,
                                             disable_bounds_checks=True),
    )
    def lookup(s_hbm, idx_hbm, out_hbm,
               idx_v, g0, g1, g2, g3, t0, t1,
               gs0, gs1, gs2, gs3, ss0, ss1):
        wid = lax.axis_index("s") * NUM_CORES + lax.axis_index("c")
        gbufs = (g0, g1, g2, g3)
        tbufs = (t0, t1)
        gsems = (gs0, gs1, gs2, gs3)
        ssems = (ss0, ss1)
        tvecs = [_iota16() + 16 * g for g in range(CHUNK // LANES)]

        pltpu.sync_copy(idx_hbm.at[wid], idx_v)

        def issue_gather(t):
            return pltpu.async_copy(
                s_hbm.at[idx_v.at[t]], gbufs[t % NBUF2], gsems[t % NBUF2])

        def issue_store(t):
            g = wid * n_chunks + t
            s_i = g // blocks_per_sent
            nb = g % blocks_per_sent
            return pltpu.async_copy(
                tbufs[t % 2], out_hbm.at[s_i, :, nb], ssems[t % 2])

        def scale_transpose(t):
            src = gbufs[t % NBUF2]
            dst = tbufs[t % 2]

            @plsc.parallel_loop(0, DIM, step=1, unroll=2)
            def _dim(d):
                dvec = jnp.broadcast_to(d, (LANES,)).astype(jnp.int32)
                r = lax.shift_right_logical(d, 3)
                dr = lax.bitwise_and(d, 7)
                for g in range(CHUNK // LANES):
                    v = plsc.load_gather(src, [tvecs[g], dvec])
                    dst[r, dr, pl.ds(16 * g, LANES)] = v * jnp.float32(SCALE)

        gathers = {}
        for t in range(min(NBUF2 - 1, n_chunks)):
            gathers[t] = issue_gather(t)
        stores = {}
        for t in range(n_chunks):
            if t + NBUF2 - 1 < n_chunks:
                gathers[t + NBUF2 - 1] = issue_gather(t + NBUF2 - 1)
            gathers[t].wait()
            if t >= 2:
                stores[t - 2].wait()
            scale_transpose(t)
            stores[t] = issue_store(t)
        stores[n_chunks - 2].wait()
        stores[n_chunks - 1].wait()

    return lookup


def kernel(inputSWE, table):
    n_sent, sent_len = inputSWE.shape
    total = n_sent * sent_len
    n_chunks = total // (NUM_WORKERS * CHUNK)
    idx = inputSWE.reshape(NUM_WORKERS, n_chunks, CHUNK).astype(jnp.int32)
    tail = table[FULL_COLS:].reshape(TAIL_ROWS, 2 * DIM)
    s_pair = _make_retile()(table.T, tail)
    s_rm = s_pair.reshape(VOCAB, DIM)
    out5 = _make_lookup(n_chunks, n_sent, sent_len)(s_rm, idx)
    out_phys = out5.transpose(0, 1, 3, 2, 4).reshape(n_sent, DIM, sent_len)
    return out_phys.transpose(0, 2, 1)


# speedup vs baseline: 1.2923x; 1.2923x over previous
import functools

import jax
import jax.numpy as jnp
from jax import lax
from jax.experimental import pallas as pl
from jax.experimental.pallas import tpu as pltpu
from jax.experimental.pallas import tpu_sc as plsc

NUM_CORES = 2
NUM_SUBCORES = 16
NUM_WORKERS = NUM_CORES * NUM_SUBCORES
LANES = 16
DIM = 64
SCALE = 8.0

CHUNK = 128
NBUF2 = 4


def _iota16():
    return lax.iota(jnp.int32, LANES)


@functools.lru_cache(maxsize=None)
def _make_lookup(n_chunks: int, n_sent: int, sent_len: int):
    mesh = plsc.VectorSubcoreMesh(core_axis_name="c", subcore_axis_name="s")
    blocks_per_sent = sent_len // CHUNK

    @functools.partial(
        pl.kernel,
        mesh=mesh,
        out_type=jax.ShapeDtypeStruct(
            (n_sent, DIM // 8, sent_len // CHUNK, 8, CHUNK), jnp.float32),
        scratch_types=(
            [pltpu.VMEM((n_chunks, CHUNK), jnp.int32)]
            + [pltpu.VMEM((CHUNK, DIM), jnp.float32)] * NBUF2
            + [pltpu.VMEM((DIM // 8, 8, CHUNK), jnp.float32)] * 2
            + [pltpu.SemaphoreType.DMA] * (NBUF2 + 2)
        ),
        compiler_params=pltpu.CompilerParams(use_tc_tiling_on_sc=False,
                                             needs_layout_passes=False,
                                             disable_bounds_checks=True),
    )
    def lookup(s_hbm, idx_hbm, out_hbm,
               idx_v, g0, g1, g2, g3, t0, t1,
               gs0, gs1, gs2, gs3, ss0, ss1):
        wid = lax.axis_index("s") * NUM_CORES + lax.axis_index("c")
        gbufs = (g0, g1, g2, g3)
        tbufs = (t0, t1)
        gsems = (gs0, gs1, gs2, gs3)
        ssems = (ss0, ss1)
        tvecs = [_iota16() + 16 * g for g in range(CHUNK // LANES)]

        pltpu.sync_copy(idx_hbm.at[wid], idx_v)

        def issue_gather(t):
            return pltpu.async_copy(
                s_hbm.at[idx_v.at[t]], gbufs[t % NBUF2], gsems[t % NBUF2])

        def issue_store(t):
            g = wid * n_chunks + t
            s_i = g // blocks_per_sent
            nb = g % blocks_per_sent
            return pltpu.async_copy(
                tbufs[t % 2], out_hbm.at[s_i, :, nb], ssems[t % 2])

        def scale_transpose(t):
            src = gbufs[t % NBUF2]
            dst = tbufs[t % 2]

            @plsc.parallel_loop(0, DIM, step=1, unroll=2)
            def _dim(d):
                dvec = jnp.broadcast_to(d, (LANES,)).astype(jnp.int32)
                r = lax.shift_right_logical(d, 3)
                dr = lax.bitwise_and(d, 7)
                for g in range(CHUNK // LANES):
                    v = plsc.load_gather(src, [tvecs[g], dvec])
                    dst[r, dr, pl.ds(16 * g, LANES)] = v * jnp.float32(SCALE)

        gathers = {}
        for t in range(min(NBUF2 - 1, n_chunks)):
            gathers[t] = issue_gather(t)
        stores = {}
        for t in range(n_chunks):
            if t + NBUF2 - 1 < n_chunks:
                gathers[t + NBUF2 - 1] = issue_gather(t + NBUF2 - 1)
            gathers[t].wait()
            if t >= 2:
                stores[t - 2].wait()
            scale_transpose(t)
            stores[t] = issue_store(t)
        stores[n_chunks - 2].wait()
        stores[n_chunks - 1].wait()

    return lookup


def kernel(inputSWE, table):
    n_sent, sent_len = inputSWE.shape
    total = n_sent * sent_len
    n_chunks = total // (NUM_WORKERS * CHUNK)
    idx = inputSWE.reshape(NUM_WORKERS, n_chunks, CHUNK).astype(jnp.int32)
    out5 = _make_lookup(n_chunks, n_sent, sent_len)(table, idx)
    out_phys = out5.transpose(0, 1, 3, 2, 4).reshape(n_sent, DIM, sent_len)
    return out_phys.transpose(0, 2, 1)


# speedup vs baseline: 1.3208x; 1.0221x over previous
import functools

import jax
import jax.numpy as jnp
from jax import lax
from jax.experimental import pallas as pl
from jax.experimental.pallas import tpu as pltpu
from jax.experimental.pallas import tpu_sc as plsc

NUM_CORES = 2
NUM_SUBCORES = 16
NUM_WORKERS = NUM_CORES * NUM_SUBCORES
CHUNK = 128
SUB = 5
BIG = SUB * CHUNK
DIM = 64
LANES = 16
SCALE = 8.0


@functools.lru_cache(maxsize=None)
def _make_lookup(n_big: int):
    mesh = plsc.VectorSubcoreMesh(core_axis_name="c", subcore_axis_name="s")
    n_idx_rows = n_big * SUB

    @functools.partial(
        pl.kernel,
        mesh=mesh,
        out_type=jax.ShapeDtypeStruct((NUM_WORKERS, n_big, BIG, DIM),
                                      jnp.float32),
        scratch_types=[
            pltpu.VMEM((n_idx_rows, CHUNK), jnp.int32),
            pltpu.VMEM((BIG, DIM), jnp.float32),
            pltpu.VMEM((BIG, DIM), jnp.float32),
            pltpu.SemaphoreType.DMA,
            pltpu.SemaphoreType.DMA,
            pltpu.SemaphoreType.DMA,
            pltpu.SemaphoreType.DMA,
        ],
        compiler_params=pltpu.CompilerParams(use_tc_tiling_on_sc=False),
    )
    def lookup(table_hbm, idx_hbm, out_hbm, idx_v, buf0, buf1,
               gs0, gs1, ss0, ss1):
        wid = lax.axis_index("s") * NUM_CORES + lax.axis_index("c")
        bufs = (buf0, buf1)
        gsems = (gs0, gs1)
        ssems = (ss0, ss1)

        pltpu.sync_copy(idx_hbm.at[wid], idx_v)

        def fire_gathers(t):
            b = t % 2
            return [
                pltpu.async_copy(
                    table_hbm.at[idx_v.at[t * SUB + k]],
                    bufs[b].at[pl.ds(k * CHUNK, CHUNK)],
                    gsems[b],
                )
                for k in range(SUB)
            ]

        pending = {0: fire_gathers(0)}
        stores = {}
        for t in range(n_big):
            b = t % 2
            if t + 1 < n_big:
                if t >= 1:
                    stores[t - 1].wait()
                pending[t + 1] = fire_gathers(t + 1)
            for c in pending[t]:
                c.wait()

            buf = bufs[b]

            @plsc.parallel_loop(0, BIG, step=1, unroll=8)
            def _scale_row(r):
                for cc in range(DIM // LANES):
                    sl = pl.ds(cc * LANES, LANES)
                    buf[r, sl] = buf[r, sl] * jnp.float32(SCALE)

            stores[t] = pltpu.async_copy(buf, out_hbm.at[wid, t], ssems[b])

        stores[n_big - 2].wait()
        stores[n_big - 1].wait()

    return lookup


def kernel(inputSWE, table):
    s, n = inputSWE.shape
    b = s * n
    n_big = b // (NUM_WORKERS * BIG)
    idx = inputSWE.reshape(NUM_WORKERS, n_big * SUB, CHUNK).astype(jnp.int32)
    out = _make_lookup(n_big)(table, idx)
    return out.reshape(s, n, DIM)
